# Initial kernel scaffold; baseline (speedup 1.0000x reference)
import jax
import jax.numpy as jnp
from jax.experimental import pallas as pl


def kernel(x, W, b):
    raise NotImplementedError("write your pallas kernel here")


# speedup vs baseline: 1.0582x; 1.0582x over previous
import functools

import jax
import jax.numpy as jnp
from jax.experimental import pallas as pl
from jax.experimental.pallas import tpu as pltpu

TOKENS = 16384
D_MODEL = 4096
N_NEURONS = 64
TOP_K = 8
BLK = 512


def _router_body(x_ref, w_ref, b_ref, gates_ref, idx_ref):
    x = x_ref[...]
    w = w_ref[...]
    logits = jax.lax.dot_general(
        x, w, (((1,), (1,)), ((), ())), preferred_element_type=jnp.float32
    )
    logits = logits + b_ref[...]

    iota = jax.lax.broadcasted_iota(jnp.int32, logits.shape, 1)
    neg_inf = jnp.float32(-jnp.inf)

    vals = []
    idxs = []
    l = logits
    for _ in range(TOP_K):
        m = jnp.max(l, axis=1, keepdims=True)
        cand = jnp.where(l == m, iota, N_NEURONS)
        am = jnp.min(cand, axis=1, keepdims=True)
        vals.append(m)
        idxs.append(am)
        l = jnp.where(iota == am, neg_inf, l)

    v = jnp.concatenate(vals, axis=1)
    e = jnp.exp(v - v[:, 0:1])
    gates_ref[...] = e / jnp.sum(e, axis=1, keepdims=True)
    idx_ref[...] = jnp.concatenate(idxs, axis=1)


@functools.partial(jax.jit, static_argnames=())
def kernel(x, W, b):
    grid = (TOKENS // BLK,)
    gates, idx = pl.pallas_call(
        _router_body,
        grid=grid,
        in_specs=[
            pl.BlockSpec((BLK, D_MODEL), lambda i: (i, 0)),
            pl.BlockSpec((N_NEURONS, D_MODEL), lambda i: (0, 0)),
            pl.BlockSpec((1, N_NEURONS), lambda i: (0, 0)),
        ],
        out_specs=[
            pl.BlockSpec((BLK, TOP_K), lambda i: (i, 0)),
            pl.BlockSpec((BLK, TOP_K), lambda i: (i, 0)),
        ],
        out_shape=[
            jax.ShapeDtypeStruct((TOKENS, TOP_K), jnp.float32),
            jax.ShapeDtypeStruct((TOKENS, TOP_K), jnp.int32),
        ],
    )(x, W, b.reshape(1, N_NEURONS))
    return gates, idx


# speedup vs baseline: 1.4731x; 1.3921x over previous
import functools

import jax
import jax.numpy as jnp
from jax.experimental import pallas as pl
from jax.experimental.pallas import tpu as pltpu

TOKENS = 16384
D_MODEL = 4096
N_NEURONS = 64
TOP_K = 8
BLK = 512


def _router_body(x_ref, w_ref, b_ref, gates_ref, idx_ref):
    x = x_ref[...]
    w = w_ref[...]
    logits = jax.lax.dot_general(
        w, x, (((1,), (1,)), ((), ())), preferred_element_type=jnp.float32
    )
    logits = logits + b_ref[...]

    iota_f = jax.lax.broadcasted_iota(jnp.int32, logits.shape, 0).astype(jnp.float32)
    neg_inf = jnp.float32(-jnp.inf)
    sentinel = jnp.float32(N_NEURONS)

    vals = []
    idxs = []
    l = logits
    for _ in range(TOP_K):
        m = jnp.max(l, axis=0, keepdims=True)
        cand = jnp.where(l == m, iota_f, sentinel)
        am = jnp.min(cand, axis=0, keepdims=True)
        vals.append(m)
        idxs.append(am)
        l = jnp.where(cand == am, neg_inf, l)

    v = jnp.concatenate(vals, axis=0)
    e = jnp.exp(v - v[0:1])
    g = e / jnp.sum(e, axis=0, keepdims=True)
    idx_f = jnp.concatenate(idxs, axis=0)

    gates_ref[...] = g.T
    idx_ref[...] = idx_f.T.astype(jnp.int32)


@functools.partial(jax.jit, static_argnames=())
def kernel(x, W, b):
    grid = (TOKENS // BLK,)
    gates, idx = pl.pallas_call(
        _router_body,
        grid=grid,
        in_specs=[
            pl.BlockSpec((BLK, D_MODEL), lambda i: (i, 0)),
            pl.BlockSpec((N_NEURONS, D_MODEL), lambda i: (0, 0)),
            pl.BlockSpec((N_NEURONS, 1), lambda i: (0, 0)),
        ],
        out_specs=[
            pl.BlockSpec((BLK, TOP_K), lambda i: (i, 0)),
            pl.BlockSpec((BLK, TOP_K), lambda i: (i, 0)),
        ],
        out_shape=[
            jax.ShapeDtypeStruct((TOKENS, TOP_K), jnp.float32),
            jax.ShapeDtypeStruct((TOKENS, TOP_K), jnp.int32),
        ],
    )(x, W, b.reshape(N_NEURONS, 1))
    return gates, idx


# speedup vs baseline: 1.5776x; 1.0709x over previous
import functools

import jax
import jax.numpy as jnp
from jax.experimental import pallas as pl
from jax.experimental.pallas import tpu as pltpu

TOKENS = 16384
D_MODEL = 4096
N_NEURONS = 64
TOP_K = 8
BLK = 1024


def _router_body(x_ref, w_ref, b_ref, gates_ref, idx_ref):
    x = x_ref[...]
    w = w_ref[...]
    logits = jax.lax.dot_general(
        w, x, (((1,), (1,)), ((), ())), preferred_element_type=jnp.float32
    )
    logits = logits + b_ref[...]

    iota_f = jax.lax.broadcasted_iota(jnp.int32, logits.shape, 0).astype(jnp.float32)
    neg_inf = jnp.float32(-jnp.inf)
    sentinel = jnp.float32(N_NEURONS)

    vals = []
    idxs = []
    l = logits
    for _ in range(TOP_K):
        m = jnp.max(l, axis=0, keepdims=True)
        cand = jnp.where(l == m, iota_f, sentinel)
        am = jnp.min(cand, axis=0, keepdims=True)
        vals.append(m)
        idxs.append(am)
        l = jnp.where(cand == am, neg_inf, l)

    v = jnp.concatenate(vals, axis=0)
    e = jnp.exp(v - v[0:1])
    g = e / jnp.sum(e, axis=0, keepdims=True)
    idx_f = jnp.concatenate(idxs, axis=0)

    gates_ref[...] = g.T
    idx_ref[...] = idx_f.T.astype(jnp.int32)


@functools.partial(jax.jit, static_argnames=())
def kernel(x, W, b):
    grid = (TOKENS // BLK,)
    gates, idx = pl.pallas_call(
        _router_body,
        grid=grid,
        in_specs=[
            pl.BlockSpec((BLK, D_MODEL), lambda i: (i, 0)),
            pl.BlockSpec((N_NEURONS, D_MODEL), lambda i: (0, 0)),
            pl.BlockSpec((N_NEURONS, 1), lambda i: (0, 0)),
        ],
        out_specs=[
            pl.BlockSpec((BLK, TOP_K), lambda i: (i, 0)),
            pl.BlockSpec((BLK, TOP_K), lambda i: (i, 0)),
        ],
        out_shape=[
            jax.ShapeDtypeStruct((TOKENS, TOP_K), jnp.float32),
            jax.ShapeDtypeStruct((TOKENS, TOP_K), jnp.int32),
        ],
    )(x, W, b.reshape(N_NEURONS, 1))
    return gates, idx
